# baseline (speedup 1.0000x reference)
import dataclasses
import functools

import jax
import jax.numpy as jnp
import numpy as np
from jax import lax
from jax.experimental import pallas as pl
from jax.experimental.pallas import tpu as pltpu
from jax.experimental.pallas import tpu_sc as plsc

N = 10000
E = 320000
D = 128
H = 8
C = D // H
ED = 16

NPAD = 10240
EPAD = 327680
NW = 32
EPW = EPAD // NW
G = 128
CHUNKS = EPW // G
NPC0 = 4992
ACCR = 5104
EPS = EPAD // 16
GS = 64
SCH = EPS // GS
NBLK = ACCR // 8
DROWS = 320

_mesh = plsc.VectorSubcoreMesh(core_axis_name="c", subcore_axis_name="s")
_HI = jax.lax.Precision.HIGHEST

_cp = pltpu.CompilerParams()
if "needs_layout_passes" in pltpu.CompilerParams.__dataclass_fields__:
    _cp = dataclasses.replace(_cp, needs_layout_passes=False)


def _qkv_body(x_ref, wq_ref, bq_ref, wk_ref, bk_ref, wv_ref, bv_ref,
              q_ref, kv_ref):
    x = x_ref[...]
    q = (x @ wq_ref[...] + bq_ref[...]) * np.float32(1.0 / np.sqrt(C))
    k = x @ wk_ref[...] + bk_ref[...]
    v = x @ wv_ref[...] + bv_ref[...]
    q_ref[...] = q
    kv_ref[:, :D] = k
    kv_ref[:, D:] = v


def _qkv(x, Wq, bq, Wk, bk, Wv, bv):
    B = 1024
    return pl.pallas_call(
        _qkv_body,
        grid=(NPAD // B,),
        in_specs=[
            pl.BlockSpec((B, D), lambda i: (i, 0)),
            pl.BlockSpec((D, D), lambda i: (0, 0)),
            pl.BlockSpec((1, D), lambda i: (0, 0)),
            pl.BlockSpec((D, D), lambda i: (0, 0)),
            pl.BlockSpec((1, D), lambda i: (0, 0)),
            pl.BlockSpec((D, D), lambda i: (0, 0)),
            pl.BlockSpec((1, D), lambda i: (0, 0)),
        ],
        out_specs=[
            pl.BlockSpec((B, D), lambda i: (i, 0)),
            pl.BlockSpec((B, 2 * D), lambda i: (i, 0)),
        ],
        out_shape=[
            jax.ShapeDtypeStruct((NPAD, D), jnp.float32),
            jax.ShapeDtypeStruct((NPAD, 2 * D), jnp.float32),
        ],
    )(x, Wq, bq, Wk, bk, Wv, bv)


@functools.partial(
    pl.kernel,
    mesh=_mesh,
    out_type=[
        jax.ShapeDtypeStruct((EPAD, D), jnp.float32),
        jax.ShapeDtypeStruct((EPAD, 2 * D), jnp.float32),
    ],
    scratch_types=[
        pltpu.VMEM((2, G), jnp.int32),
        pltpu.VMEM((2, G), jnp.int32),
        pltpu.VMEM((2 * G, D), jnp.float32),
        pltpu.VMEM((2 * G, 2 * D), jnp.float32),
        pltpu.SemaphoreType.DMA,
        pltpu.SemaphoreType.DMA,
        pltpu.SemaphoreType.DMA,
        pltpu.SemaphoreType.DMA,
    ],
)
def _sc_gather(q_hbm, kv_hbm, src_hbm, dst_hbm, qg_hbm, kvg_hbm,
               sidx_v, didx_v, qrow_v, kvrow_v, semg0, semg1, semw0, semw1):
    wid = lax.axis_index("s") * 2 + lax.axis_index("c")
    base_w = wid * EPW
    semg = (semg0, semg1)
    semw = (semw0, semw1)

    def fetch(ci, slot):
        base = base_w + ci * G
        pltpu.sync_copy(src_hbm.at[pl.ds(base, G)], sidx_v.at[slot])
        pltpu.sync_copy(dst_hbm.at[pl.ds(base, G)], didx_v.at[slot])
        pltpu.async_copy(kv_hbm.at[sidx_v.at[slot]],
                         kvrow_v.at[pl.ds(slot * G, G)], semg[slot])
        pltpu.async_copy(q_hbm.at[didx_v.at[slot]],
                         qrow_v.at[pl.ds(slot * G, G)], semg[slot])

    def drain(ci, slot):
        base = base_w + ci * G
        pltpu.make_async_copy(kv_hbm.at[sidx_v.at[slot]],
                              kvrow_v.at[pl.ds(slot * G, G)],
                              semg[slot]).wait()
        pltpu.make_async_copy(q_hbm.at[didx_v.at[slot]],
                              qrow_v.at[pl.ds(slot * G, G)],
                              semg[slot]).wait()
        pltpu.async_copy(kvrow_v.at[pl.ds(slot * G, G)],
                         kvg_hbm.at[pl.ds(base, G)], semw[slot])
        pltpu.async_copy(qrow_v.at[pl.ds(slot * G, G)],
                         qg_hbm.at[pl.ds(base, G)], semw[slot])

    def wait_wb(ci, slot):
        base = base_w + ci * G
        pltpu.make_async_copy(kvrow_v.at[pl.ds(slot * G, G)],
                              kvg_hbm.at[pl.ds(base, G)], semw[slot]).wait()
        pltpu.make_async_copy(qrow_v.at[pl.ds(slot * G, G)],
                              qg_hbm.at[pl.ds(base, G)], semw[slot]).wait()

    fetch(0, 0)
    fetch(1, 1)

    @pl.loop(0, CHUNKS - 2, step=2)
    def _(ci):
        drain(ci, 0)
        wait_wb(ci, 0)
        fetch(ci + 2, 0)
        drain(ci + 1, 1)
        wait_wb(ci + 1, 1)
        fetch(ci + 3, 1)

    drain(CHUNKS - 2, 0)
    wait_wb(CHUNKS - 2, 0)
    drain(CHUNKS - 1, 1)
    wait_wb(CHUNKS - 1, 1)


def _edge_body(qg_ref, kvg_ref, ea_ref, we_ref, num_ref, ex_ref):
    q = qg_ref[...]
    k = kvg_ref[:, :D]
    v = kvg_ref[:, D:]
    e = jax.lax.dot_general(ea_ref[...], we_ref[...],
                            (((1,), (0,)), ((), ())), precision=_HI)
    k_j = k + e
    v_j = v + e
    p = q * k_j
    r8 = jax.lax.broadcasted_iota(jnp.int32, (D, 16), 0) // C
    c8 = jax.lax.broadcasted_iota(jnp.int32, (D, 16), 1)
    s8 = ((r8 == c8) & (c8 < H)).astype(jnp.float32)
    alpha8 = jax.lax.dot_general(p, s8, (((1,), (0,)), ((), ())),
                                 precision=_HI)
    m8 = (jax.lax.broadcasted_iota(jnp.int32, alpha8.shape, 1) < H)
    ex8 = jnp.where(m8, jnp.exp(alpha8), 0.0)
    rt = jax.lax.broadcasted_iota(jnp.int32, (16, D), 0)
    ct = jax.lax.broadcasted_iota(jnp.int32, (16, D), 1) // C
    s8t = (rt == ct).astype(jnp.float32)
    ex128 = jax.lax.dot_general(ex8, s8t, (((1,), (0,)), ((), ())),
                                precision=_HI)
    num_ref[...] = ex128 * v_j
    ex_ref[...] = ex8


def _edge_math(qg, kvg, ea, We):
    B = 2048
    return pl.pallas_call(
        _edge_body,
        grid=(EPAD // B,),
        in_specs=[
            pl.BlockSpec((B, D), lambda i: (i, 0)),
            pl.BlockSpec((B, 2 * D), lambda i: (i, 0)),
            pl.BlockSpec((B, ED), lambda i: (i, 0)),
            pl.BlockSpec((ED, D), lambda i: (0, 0)),
        ],
        out_specs=[
            pl.BlockSpec((B, D), lambda i: (i, 0)),
            pl.BlockSpec((B, 16), lambda i: (i, 0)),
        ],
        out_shape=[
            jax.ShapeDtypeStruct((EPAD, D), jnp.float32),
            jax.ShapeDtypeStruct((EPAD, 16), jnp.float32),
        ],
    )(qg, kvg, ea, We)


@functools.partial(
    pl.kernel,
    mesh=_mesh,
    out_type=[
        jax.ShapeDtypeStruct((2, ACCR, D), jnp.float32),
        jax.ShapeDtypeStruct((2, DROWS, D), jnp.float32),
    ],
    scratch_types=[
        pltpu.VMEM((2, GS), jnp.int32),
        pltpu.VMEM((2, GS), jnp.int32),
        pltpu.VMEM((64,), jnp.int32),
        pltpu.VMEM((2 * GS, D), jnp.float32),
        pltpu.VMEM((2 * GS, 16), jnp.float32),
        pltpu.VMEM((8, D), jnp.float32),
        pltpu.VMEM((DROWS, D), jnp.float32),
        pltpu.VMEM_SHARED((ACCR, D), jnp.float32),
        pltpu.VMEM_SHARED((DROWS, D), jnp.float32),
        pltpu.SemaphoreType.DMA,
        pltpu.SemaphoreType.DMA,
        pltpu.SemaphoreType.DMA,
        pltpu.SemaphoreType.DMA,
        pltpu.SemaphoreType.DMA,
    ],
    compiler_params=_cp,
)
def _sc_scatter(num_hbm, ex_hbm, dst_hbm, outn_hbm, outd_hbm, didx_v, lidx_v,
                midx_v, nrows_v, xrows_v, zbuf_v, den_v, accn_sh, accd_sh,
                semf0, semf1, semn0, semn1, semd):
    cid = lax.axis_index("c")
    sid = lax.axis_index("s")
    base_node = cid * NPC0
    limit = jnp.where(cid == 0, NPC0, ACCR)
    trash = jnp.where(cid == 0, NPC0 + 8, ACCR - 1)
    iota16 = lax.iota(jnp.int32, 16)

    @pl.loop(0, 8)
    def _(rr):
        for cc in range(0, D, 16):
            zbuf_v[rr, pl.ds(cc, 16)] = jnp.zeros((16,), jnp.float32)

    @pl.loop(0, DROWS)
    def _(rr):
        for cc in range(0, D, 16):
            den_v[rr, pl.ds(cc, 16)] = jnp.zeros((16,), jnp.float32)

    for j in range((NBLK + 15) // 16):
        blk = j * 16 + sid

        @pl.when(blk < NBLK)
        def _():
            pltpu.sync_copy(zbuf_v, accn_sh.at[pl.ds(blk * 8, 8)])
    for j in range((DROWS // 8 + 15) // 16):
        blk = j * 16 + sid

        @pl.when(blk < DROWS // 8)
        def _():
            pltpu.sync_copy(zbuf_v, accd_sh.at[pl.ds(blk * 8, 8)])
    plsc.subcore_barrier()

    semf = (semf0, semf1)
    semn = (semn0, semn1)

    def fetch(ci, slot):
        base = sid * EPS + ci * GS
        pltpu.async_copy(dst_hbm.at[pl.ds(base, GS)], didx_v.at[slot],
                         semf[slot])
        pltpu.async_copy(num_hbm.at[pl.ds(base, GS)],
                         nrows_v.at[pl.ds(slot * GS, GS)], semf[slot])
        pltpu.async_copy(ex_hbm.at[pl.ds(base, GS)],
                         xrows_v.at[pl.ds(slot * GS, GS)], semf[slot])

    def proc(ci, slot):
        base = sid * EPS + ci * GS
        pltpu.make_async_copy(dst_hbm.at[pl.ds(base, GS)], didx_v.at[slot],
                              semf[slot]).wait()
        pltpu.make_async_copy(num_hbm.at[pl.ds(base, GS)],
                              nrows_v.at[pl.ds(slot * GS, GS)],
                              semf[slot]).wait()
        pltpu.make_async_copy(ex_hbm.at[pl.ds(base, GS)],
                              xrows_v.at[pl.ds(slot * GS, GS)],
                              semf[slot]).wait()
        for i in range(GS // 16):
            d = didx_v[slot, pl.ds(i * 16, 16)]
            rel = d - base_node
            ok = (rel >= 0) & (rel < limit)
            lidx_v[slot, pl.ds(i * 16, 16)] = jnp.where(ok, rel, trash)
        a = pltpu.async_copy(nrows_v.at[pl.ds(slot * GS, GS)],
                             accn_sh.at[lidx_v.at[slot]], semn[slot],
                             add=True)

        @pl.loop(0, GS // 16)
        def _(i):
            lv = lidx_v[slot, pl.ds(i * 16, 16)]
            for j in range(16):
                f = lv[j] * 8
                fl = f + iota16
                row = lax.shift_right_logical(fl, 7)
                col = lax.bitwise_and(fl, 127)
                val = xrows_v[slot * GS + i * 16 + j, :]
                plsc.addupdate_scatter(den_v, [row, col], val)

        a.wait()

    fetch(0, 0)
    fetch(1, 1)

    @pl.loop(0, SCH - 2, step=2)
    def _(ci):
        proc(ci, 0)
        fetch(ci + 2, 0)
        proc(ci + 1, 1)
        fetch(ci + 3, 1)

    proc(SCH - 2, 0)
    proc(SCH - 1, 1)

    for c in range(DROWS // 64):
        for i in range(4):
            midx_v[pl.ds(i * 16, 16)] = iota16 + (c * 64 + i * 16)
        pltpu.async_copy(den_v.at[pl.ds(c * 64, 64)], accd_sh.at[midx_v],
                         semd, add=True).wait()

    plsc.subcore_barrier()
    for j in range((NBLK + 15) // 16):
        blk = j * 16 + sid

        @pl.when(blk < NBLK)
        def _():
            pltpu.sync_copy(accn_sh.at[pl.ds(blk * 8, 8)],
                            outn_hbm.at[cid].at[pl.ds(blk * 8, 8)])
    for j in range((DROWS // 8 + 15) // 16):
        blk = j * 16 + sid

        @pl.when(blk < DROWS // 8)
        def _():
            pltpu.sync_copy(accd_sh.at[pl.ds(blk * 8, 8)],
                            outd_hbm.at[cid].at[pl.ds(blk * 8, 8)])


def _tail_body(num_ref, den8_ref, x_ref, wsk_ref, bsk_ref, g1_ref,
               b1n_ref, w1_ref, b1_ref, w2_ref, b2_ref, g2_ref, b2n_ref,
               o_ref):
    num = num_ref[...]
    den8 = den8_ref[...]
    r8 = jax.lax.broadcasted_iota(jnp.int32, (8, D), 0)
    c8 = jax.lax.broadcasted_iota(jnp.int32, (8, D), 1) // C
    s8t = (r8 == c8).astype(jnp.float32)
    den128 = jax.lax.dot_general(den8, s8t, (((1,), (0,)), ((), ())),
                                 precision=_HI)
    agg = num / (den128 + np.float32(1e-16))
    x = x_ref[...]
    conv = agg + x @ wsk_ref[...] + bsk_ref[...]
    y = conv + x
    m = jnp.mean(y, axis=-1, keepdims=True)
    var = jnp.mean((y - m) ** 2, axis=-1, keepdims=True)
    h = (y - m) / jnp.sqrt(var + 1e-5) * g1_ref[...] + b1n_ref[...]
    z = h @ w1_ref[...] + b1_ref[...]
    gel = z * 0.5 * (1.0 + jax.lax.erf(z * np.float32(1.0 / np.sqrt(2.0))))
    f = gel @ w2_ref[...] + b2_ref[...]
    y2 = f + h
    m2 = jnp.mean(y2, axis=-1, keepdims=True)
    var2 = jnp.mean((y2 - m2) ** 2, axis=-1, keepdims=True)
    o_ref[...] = (y2 - m2) / jnp.sqrt(var2 + 1e-5) * g2_ref[...] + b2n_ref[...]


def _tail(num, den8, x, Wskip, bskip, ln1_g, ln1_b, W1, b1, W2, b2,
          ln2_g, ln2_b):
    B = 1024
    return pl.pallas_call(
        _tail_body,
        grid=(NPAD // B,),
        in_specs=[
            pl.BlockSpec((B, D), lambda i: (i, 0)),
            pl.BlockSpec((B, 8), lambda i: (i, 0)),
            pl.BlockSpec((B, D), lambda i: (i, 0)),
            pl.BlockSpec((D, D), lambda i: (0, 0)),
            pl.BlockSpec((1, D), lambda i: (0, 0)),
            pl.BlockSpec((1, D), lambda i: (0, 0)),
            pl.BlockSpec((1, D), lambda i: (0, 0)),
            pl.BlockSpec((D, 4 * D), lambda i: (0, 0)),
            pl.BlockSpec((1, 4 * D), lambda i: (0, 0)),
            pl.BlockSpec((4 * D, D), lambda i: (0, 0)),
            pl.BlockSpec((1, D), lambda i: (0, 0)),
            pl.BlockSpec((1, D), lambda i: (0, 0)),
            pl.BlockSpec((1, D), lambda i: (0, 0)),
        ],
        out_specs=pl.BlockSpec((B, D), lambda i: (i, 0)),
        out_shape=jax.ShapeDtypeStruct((NPAD, D), jnp.float32),
    )(num, den8, x, Wskip, bskip, ln1_g, ln1_b,
      W1, b1, W2, b2, ln2_g, ln2_b)


def kernel(x, edge_index, edge_attr, Wq, bq, Wk, bk, Wv, bv, We, Wskip, bskip,
           ln1_g, ln1_b, ln2_g, ln2_b, W1, b1, W2, b2):
    xp = jnp.pad(x, ((0, NPAD - N), (0, 0)))
    q, kv = _qkv(xp, Wq, bq.reshape(1, D), Wk, bk.reshape(1, D),
                 Wv, bv.reshape(1, D))

    src = edge_index[0]
    dst = edge_index[1]
    srcp = jnp.concatenate([src, jnp.zeros((EPAD - E,), src.dtype)]).astype(jnp.int32)
    dstp = jnp.concatenate([dst, jnp.full((EPAD - E,), N, dst.dtype)]).astype(jnp.int32)

    qg, kvg = _sc_gather(q, kv, srcp, dstp)

    eap = jnp.pad(edge_attr, ((0, EPAD - E), (0, 0)))
    nume, exe = _edge_math(qg, kvg, eap, We)

    pn, pd = _sc_scatter(nume, exe, dstp)
    num = jnp.pad(jnp.concatenate([pn[0, :NPC0], pn[1]], axis=0),
                  ((0, NPAD - NPC0 - ACCR), (0, 0)))
    pd8 = pd.reshape(2, DROWS * D // 8, 8)
    den8 = jnp.pad(jnp.concatenate([pd8[0, :NPC0], pd8[1, :ACCR]], axis=0),
                   ((0, NPAD - NPC0 - ACCR), (0, 0)))

    out = _tail(num, den8, xp, Wskip, bskip.reshape(1, D),
                ln1_g.reshape(1, D), ln1_b.reshape(1, D),
                W1, b1.reshape(1, 4 * D), W2, b2.reshape(1, D),
                ln2_g.reshape(1, D), ln2_b.reshape(1, D))
    return out[:N]


# speedup vs baseline: 1.0064x; 1.0064x over previous
import dataclasses
import functools

import jax
import jax.numpy as jnp
import numpy as np
from jax import lax
from jax.experimental import pallas as pl
from jax.experimental.pallas import tpu as pltpu
from jax.experimental.pallas import tpu_sc as plsc

N = 10000
E = 320000
D = 128
H = 8
C = D // H
ED = 16

NPAD = 10240
EPAD = 327680
NW = 32
EPW = EPAD // NW
G = 128
CHUNKS = EPW // G
NPC0 = 4992
ACCR = 5104
EPS = EPAD // 16
GS = 64
SCH = EPS // GS
NBLK = ACCR // 8
DROWS = 320

_mesh = plsc.VectorSubcoreMesh(core_axis_name="c", subcore_axis_name="s")
_HI = jax.lax.Precision.HIGHEST

_cp = pltpu.CompilerParams()
if "needs_layout_passes" in pltpu.CompilerParams.__dataclass_fields__:
    _cp = dataclasses.replace(_cp, needs_layout_passes=False)


def _qkv_body(x_ref, wq_ref, bq_ref, wk_ref, bk_ref, wv_ref, bv_ref,
              q_ref, kv_ref):
    x = x_ref[...]
    q = (x @ wq_ref[...] + bq_ref[...]) * np.float32(1.0 / np.sqrt(C))
    k = x @ wk_ref[...] + bk_ref[...]
    v = x @ wv_ref[...] + bv_ref[...]
    q_ref[...] = q
    kv_ref[:, :D] = k
    kv_ref[:, D:] = v


def _qkv(x, Wq, bq, Wk, bk, Wv, bv):
    B = 1024
    return pl.pallas_call(
        _qkv_body,
        grid=(NPAD // B,),
        in_specs=[
            pl.BlockSpec((B, D), lambda i: (i, 0)),
            pl.BlockSpec((D, D), lambda i: (0, 0)),
            pl.BlockSpec((1, D), lambda i: (0, 0)),
            pl.BlockSpec((D, D), lambda i: (0, 0)),
            pl.BlockSpec((1, D), lambda i: (0, 0)),
            pl.BlockSpec((D, D), lambda i: (0, 0)),
            pl.BlockSpec((1, D), lambda i: (0, 0)),
        ],
        out_specs=[
            pl.BlockSpec((B, D), lambda i: (i, 0)),
            pl.BlockSpec((B, 2 * D), lambda i: (i, 0)),
        ],
        out_shape=[
            jax.ShapeDtypeStruct((NPAD, D), jnp.float32),
            jax.ShapeDtypeStruct((NPAD, 2 * D), jnp.float32),
        ],
    )(x, Wq, bq, Wk, bk, Wv, bv)


@functools.partial(
    pl.kernel,
    mesh=_mesh,
    out_type=[
        jax.ShapeDtypeStruct((EPAD, D), jnp.float32),
        jax.ShapeDtypeStruct((EPAD, 2 * D), jnp.float32),
    ],
    scratch_types=[
        pltpu.VMEM((2, G), jnp.int32),
        pltpu.VMEM((2, G), jnp.int32),
        pltpu.VMEM((2 * G, D), jnp.float32),
        pltpu.VMEM((2 * G, 2 * D), jnp.float32),
        pltpu.SemaphoreType.DMA,
        pltpu.SemaphoreType.DMA,
        pltpu.SemaphoreType.DMA,
        pltpu.SemaphoreType.DMA,
    ],
)
def _sc_gather(q_hbm, kv_hbm, src_hbm, dst_hbm, qg_hbm, kvg_hbm,
               sidx_v, didx_v, qrow_v, kvrow_v, semg0, semg1, semw0, semw1):
    wid = lax.axis_index("s") * 2 + lax.axis_index("c")
    base_w = wid * EPW
    semg = (semg0, semg1)
    semw = (semw0, semw1)

    def fetch(ci, slot):
        base = base_w + ci * G
        pltpu.sync_copy(src_hbm.at[pl.ds(base, G)], sidx_v.at[slot])
        pltpu.sync_copy(dst_hbm.at[pl.ds(base, G)], didx_v.at[slot])
        pltpu.async_copy(kv_hbm.at[sidx_v.at[slot]],
                         kvrow_v.at[pl.ds(slot * G, G)], semg[slot])
        pltpu.async_copy(q_hbm.at[didx_v.at[slot]],
                         qrow_v.at[pl.ds(slot * G, G)], semg[slot])

    def drain(ci, slot):
        base = base_w + ci * G
        pltpu.make_async_copy(kv_hbm.at[sidx_v.at[slot]],
                              kvrow_v.at[pl.ds(slot * G, G)],
                              semg[slot]).wait()
        pltpu.make_async_copy(q_hbm.at[didx_v.at[slot]],
                              qrow_v.at[pl.ds(slot * G, G)],
                              semg[slot]).wait()
        pltpu.async_copy(kvrow_v.at[pl.ds(slot * G, G)],
                         kvg_hbm.at[pl.ds(base, G)], semw[slot])
        pltpu.async_copy(qrow_v.at[pl.ds(slot * G, G)],
                         qg_hbm.at[pl.ds(base, G)], semw[slot])

    def wait_wb(ci, slot):
        base = base_w + ci * G
        pltpu.make_async_copy(kvrow_v.at[pl.ds(slot * G, G)],
                              kvg_hbm.at[pl.ds(base, G)], semw[slot]).wait()
        pltpu.make_async_copy(qrow_v.at[pl.ds(slot * G, G)],
                              qg_hbm.at[pl.ds(base, G)], semw[slot]).wait()

    fetch(0, 0)
    fetch(1, 1)

    @pl.loop(0, CHUNKS - 2, step=2)
    def _(ci):
        drain(ci, 0)
        wait_wb(ci, 0)
        fetch(ci + 2, 0)
        drain(ci + 1, 1)
        wait_wb(ci + 1, 1)
        fetch(ci + 3, 1)

    drain(CHUNKS - 2, 0)
    wait_wb(CHUNKS - 2, 0)
    drain(CHUNKS - 1, 1)
    wait_wb(CHUNKS - 1, 1)


def _edge_body(qg_ref, kvg_ref, ea_ref, we_ref, num_ref, ex_ref):
    q = qg_ref[...]
    k = kvg_ref[:, :D]
    v = kvg_ref[:, D:]
    e = jax.lax.dot_general(ea_ref[...], we_ref[...],
                            (((1,), (0,)), ((), ())), precision=_HI)
    k_j = k + e
    v_j = v + e
    p = q * k_j
    r = jax.lax.broadcasted_iota(jnp.int32, (D, D), 0) // C
    c2 = jax.lax.broadcasted_iota(jnp.int32, (D, D), 1) // C
    ss = (r == c2).astype(jnp.float32)
    alpha128 = jax.lax.dot_general(p, ss, (((1,), (0,)), ((), ())),
                                   precision=_HI)
    ex128 = jnp.exp(alpha128)
    num_ref[...] = ex128 * v_j
    r8 = jax.lax.broadcasted_iota(jnp.int32, (D, 16), 0) // C
    c8 = jax.lax.broadcasted_iota(jnp.int32, (D, 16), 1)
    s8 = ((r8 == c8) & (c8 < H)).astype(jnp.float32) * np.float32(1.0 / C)
    ex_ref[...] = jax.lax.dot_general(ex128, s8, (((1,), (0,)), ((), ())),
                                      precision=_HI)


def _edge_math(qg, kvg, ea, We):
    B = 2048
    return pl.pallas_call(
        _edge_body,
        grid=(EPAD // B,),
        in_specs=[
            pl.BlockSpec((B, D), lambda i: (i, 0)),
            pl.BlockSpec((B, 2 * D), lambda i: (i, 0)),
            pl.BlockSpec((B, ED), lambda i: (i, 0)),
            pl.BlockSpec((ED, D), lambda i: (0, 0)),
        ],
        out_specs=[
            pl.BlockSpec((B, D), lambda i: (i, 0)),
            pl.BlockSpec((B, 16), lambda i: (i, 0)),
        ],
        out_shape=[
            jax.ShapeDtypeStruct((EPAD, D), jnp.float32),
            jax.ShapeDtypeStruct((EPAD, 16), jnp.float32),
        ],
    )(qg, kvg, ea, We)


@functools.partial(
    pl.kernel,
    mesh=_mesh,
    out_type=[
        jax.ShapeDtypeStruct((2, ACCR, D), jnp.float32),
        jax.ShapeDtypeStruct((2, DROWS, D), jnp.float32),
    ],
    scratch_types=[
        pltpu.VMEM((2, GS), jnp.int32),
        pltpu.VMEM((2, GS), jnp.int32),
        pltpu.VMEM((64,), jnp.int32),
        pltpu.VMEM((2 * GS, D), jnp.float32),
        pltpu.VMEM((2 * GS, 16), jnp.float32),
        pltpu.VMEM((8, D), jnp.float32),
        pltpu.VMEM((DROWS, D), jnp.float32),
        pltpu.VMEM_SHARED((ACCR, D), jnp.float32),
        pltpu.VMEM_SHARED((DROWS, D), jnp.float32),
        pltpu.SemaphoreType.DMA,
        pltpu.SemaphoreType.DMA,
        pltpu.SemaphoreType.DMA,
        pltpu.SemaphoreType.DMA,
        pltpu.SemaphoreType.DMA,
    ],
    compiler_params=_cp,
)
def _sc_scatter(num_hbm, ex_hbm, dst_hbm, outn_hbm, outd_hbm, didx_v, lidx_v,
                midx_v, nrows_v, xrows_v, zbuf_v, den_v, accn_sh, accd_sh,
                semf0, semf1, semn0, semn1, semd):
    cid = lax.axis_index("c")
    sid = lax.axis_index("s")
    base_node = cid * NPC0
    limit = jnp.where(cid == 0, NPC0, ACCR)
    trash = jnp.where(cid == 0, NPC0 + 8, ACCR - 1)
    iota16 = lax.iota(jnp.int32, 16)

    @pl.loop(0, 8)
    def _(rr):
        for cc in range(0, D, 16):
            zbuf_v[rr, pl.ds(cc, 16)] = jnp.zeros((16,), jnp.float32)

    @pl.loop(0, DROWS)
    def _(rr):
        for cc in range(0, D, 16):
            den_v[rr, pl.ds(cc, 16)] = jnp.zeros((16,), jnp.float32)

    for j in range((NBLK + 15) // 16):
        blk = j * 16 + sid

        @pl.when(blk < NBLK)
        def _():
            pltpu.sync_copy(zbuf_v, accn_sh.at[pl.ds(blk * 8, 8)])
    for j in range((DROWS // 8 + 15) // 16):
        blk = j * 16 + sid

        @pl.when(blk < DROWS // 8)
        def _():
            pltpu.sync_copy(zbuf_v, accd_sh.at[pl.ds(blk * 8, 8)])
    plsc.subcore_barrier()

    semf = (semf0, semf1)
    semn = (semn0, semn1)

    def fetch(ci, slot):
        base = sid * EPS + ci * GS
        pltpu.async_copy(dst_hbm.at[pl.ds(base, GS)], didx_v.at[slot],
                         semf[slot])
        pltpu.async_copy(num_hbm.at[pl.ds(base, GS)],
                         nrows_v.at[pl.ds(slot * GS, GS)], semf[slot])
        pltpu.async_copy(ex_hbm.at[pl.ds(base, GS)],
                         xrows_v.at[pl.ds(slot * GS, GS)], semf[slot])

    def proc(ci, slot):
        base = sid * EPS + ci * GS
        pltpu.make_async_copy(dst_hbm.at[pl.ds(base, GS)], didx_v.at[slot],
                              semf[slot]).wait()
        pltpu.make_async_copy(num_hbm.at[pl.ds(base, GS)],
                              nrows_v.at[pl.ds(slot * GS, GS)],
                              semf[slot]).wait()
        pltpu.make_async_copy(ex_hbm.at[pl.ds(base, GS)],
                              xrows_v.at[pl.ds(slot * GS, GS)],
                              semf[slot]).wait()
        for i in range(GS // 16):
            d = didx_v[slot, pl.ds(i * 16, 16)]
            rel = d - base_node
            ok = (rel >= 0) & (rel < limit)
            lidx_v[slot, pl.ds(i * 16, 16)] = jnp.where(ok, rel, trash)
        a = pltpu.async_copy(nrows_v.at[pl.ds(slot * GS, GS)],
                             accn_sh.at[lidx_v.at[slot]], semn[slot],
                             add=True)

        @pl.loop(0, GS // 16)
        def _(i):
            lv = lidx_v[slot, pl.ds(i * 16, 16)]
            for j in range(16):
                f = lv[j] * 8
                fl = f + iota16
                row = lax.shift_right_logical(fl, 7)
                col = lax.bitwise_and(fl, 127)
                val = xrows_v[slot * GS + i * 16 + j, :]
                plsc.addupdate_scatter(den_v, [row, col], val)

        a.wait()

    fetch(0, 0)
    fetch(1, 1)

    @pl.loop(0, SCH - 2, step=2)
    def _(ci):
        proc(ci, 0)
        fetch(ci + 2, 0)
        proc(ci + 1, 1)
        fetch(ci + 3, 1)

    proc(SCH - 2, 0)
    proc(SCH - 1, 1)

    for c in range(DROWS // 64):
        for i in range(4):
            midx_v[pl.ds(i * 16, 16)] = iota16 + (c * 64 + i * 16)
        pltpu.async_copy(den_v.at[pl.ds(c * 64, 64)], accd_sh.at[midx_v],
                         semd, add=True).wait()

    plsc.subcore_barrier()
    for j in range((NBLK + 15) // 16):
        blk = j * 16 + sid

        @pl.when(blk < NBLK)
        def _():
            pltpu.sync_copy(accn_sh.at[pl.ds(blk * 8, 8)],
                            outn_hbm.at[cid].at[pl.ds(blk * 8, 8)])
    for j in range((DROWS // 8 + 15) // 16):
        blk = j * 16 + sid

        @pl.when(blk < DROWS // 8)
        def _():
            pltpu.sync_copy(accd_sh.at[pl.ds(blk * 8, 8)],
                            outd_hbm.at[cid].at[pl.ds(blk * 8, 8)])


def _tail_body(num_ref, den8_ref, x_ref, wsk_ref, bsk_ref, g1_ref,
               b1n_ref, w1_ref, b1_ref, w2_ref, b2_ref, g2_ref, b2n_ref,
               o_ref):
    num = num_ref[...]
    den8 = den8_ref[...]
    r8 = jax.lax.broadcasted_iota(jnp.int32, (8, D), 0)
    c8 = jax.lax.broadcasted_iota(jnp.int32, (8, D), 1) // C
    s8t = (r8 == c8).astype(jnp.float32)
    den128 = jax.lax.dot_general(den8, s8t, (((1,), (0,)), ((), ())),
                                 precision=_HI)
    agg = num / (den128 + np.float32(1e-16))
    x = x_ref[...]
    conv = agg + x @ wsk_ref[...] + bsk_ref[...]
    y = conv + x
    m = jnp.mean(y, axis=-1, keepdims=True)
    var = jnp.mean((y - m) ** 2, axis=-1, keepdims=True)
    h = (y - m) / jnp.sqrt(var + 1e-5) * g1_ref[...] + b1n_ref[...]
    z = h @ w1_ref[...] + b1_ref[...]
    gel = z * 0.5 * (1.0 + jax.lax.erf(z * np.float32(1.0 / np.sqrt(2.0))))
    f = gel @ w2_ref[...] + b2_ref[...]
    y2 = f + h
    m2 = jnp.mean(y2, axis=-1, keepdims=True)
    var2 = jnp.mean((y2 - m2) ** 2, axis=-1, keepdims=True)
    o_ref[...] = (y2 - m2) / jnp.sqrt(var2 + 1e-5) * g2_ref[...] + b2n_ref[...]


def _tail(num, den8, x, Wskip, bskip, ln1_g, ln1_b, W1, b1, W2, b2,
          ln2_g, ln2_b):
    B = 1024
    return pl.pallas_call(
        _tail_body,
        grid=(NPAD // B,),
        in_specs=[
            pl.BlockSpec((B, D), lambda i: (i, 0)),
            pl.BlockSpec((B, 8), lambda i: (i, 0)),
            pl.BlockSpec((B, D), lambda i: (i, 0)),
            pl.BlockSpec((D, D), lambda i: (0, 0)),
            pl.BlockSpec((1, D), lambda i: (0, 0)),
            pl.BlockSpec((1, D), lambda i: (0, 0)),
            pl.BlockSpec((1, D), lambda i: (0, 0)),
            pl.BlockSpec((D, 4 * D), lambda i: (0, 0)),
            pl.BlockSpec((1, 4 * D), lambda i: (0, 0)),
            pl.BlockSpec((4 * D, D), lambda i: (0, 0)),
            pl.BlockSpec((1, D), lambda i: (0, 0)),
            pl.BlockSpec((1, D), lambda i: (0, 0)),
            pl.BlockSpec((1, D), lambda i: (0, 0)),
        ],
        out_specs=pl.BlockSpec((B, D), lambda i: (i, 0)),
        out_shape=jax.ShapeDtypeStruct((NPAD, D), jnp.float32),
    )(num, den8, x, Wskip, bskip, ln1_g, ln1_b,
      W1, b1, W2, b2, ln2_g, ln2_b)


def kernel(x, edge_index, edge_attr, Wq, bq, Wk, bk, Wv, bv, We, Wskip, bskip,
           ln1_g, ln1_b, ln2_g, ln2_b, W1, b1, W2, b2):
    xp = jnp.pad(x, ((0, NPAD - N), (0, 0)))
    q, kv = _qkv(xp, Wq, bq.reshape(1, D), Wk, bk.reshape(1, D),
                 Wv, bv.reshape(1, D))

    src = edge_index[0]
    dst = edge_index[1]
    srcp = jnp.concatenate([src, jnp.zeros((EPAD - E,), src.dtype)]).astype(jnp.int32)
    dstp = jnp.concatenate([dst, jnp.full((EPAD - E,), N, dst.dtype)]).astype(jnp.int32)

    qg, kvg = _sc_gather(q, kv, srcp, dstp)

    eap = jnp.pad(edge_attr, ((0, EPAD - E), (0, 0)))
    nume, exe = _edge_math(qg, kvg, eap, We)

    pn, pd = _sc_scatter(nume, exe, dstp)
    num = jnp.pad(jnp.concatenate([pn[0, :NPC0], pn[1]], axis=0),
                  ((0, NPAD - NPC0 - ACCR), (0, 0)))
    pd8 = pd.reshape(2, DROWS * D // 8, 8)
    den8 = jnp.pad(jnp.concatenate([pd8[0, :NPC0], pd8[1, :ACCR]], axis=0),
                   ((0, NPAD - NPC0 - ACCR), (0, 0)))

    out = _tail(num, den8, xp, Wskip, bskip.reshape(1, D),
                ln1_g.reshape(1, D), ln1_b.reshape(1, D),
                W1, b1.reshape(1, 4 * D), W2, b2.reshape(1, D),
                ln2_g.reshape(1, D), ln2_b.reshape(1, D))
    return out[:N]


# speedup vs baseline: 1.1985x; 1.1908x over previous
import dataclasses
import functools

import jax
import jax.numpy as jnp
import numpy as np
from jax import lax
from jax.experimental import pallas as pl
from jax.experimental.pallas import tpu as pltpu
from jax.experimental.pallas import tpu_sc as plsc

N = 10000
E = 320000
D = 128
H = 8
C = D // H
ED = 16

NPAD = 10240
EPAD = 327680
EH = EPAD // 2
NW = 32
EPW = EH // NW
G = 128
CHUNKS = EPW // G
NPC0 = 4992
ACCR = 5104
EPS = EH // 16
GS = 64
SCH = EPS // GS
NBLK = ACCR // 8
DROWS = 320

_mesh = plsc.VectorSubcoreMesh(core_axis_name="c", subcore_axis_name="s")
_HI = jax.lax.Precision.HIGHEST

_cp = pltpu.CompilerParams()
if "needs_layout_passes" in pltpu.CompilerParams.__dataclass_fields__:
    _cp = dataclasses.replace(_cp, needs_layout_passes=False)


def _qkv_body(x_ref, wq_ref, bq_ref, wk_ref, bk_ref, wv_ref, bv_ref,
              q_ref, kv_ref):
    x = x_ref[...]
    q = (x @ wq_ref[...] + bq_ref[...]) * np.float32(1.0 / np.sqrt(C))
    k = x @ wk_ref[...] + bk_ref[...]
    v = x @ wv_ref[...] + bv_ref[...]
    q_ref[...] = q
    kv_ref[:, :D] = k
    kv_ref[:, D:] = v


def _qkv(x, Wq, bq, Wk, bk, Wv, bv):
    B = 1024
    return pl.pallas_call(
        _qkv_body,
        grid=(NPAD // B,),
        in_specs=[
            pl.BlockSpec((B, D), lambda i: (i, 0)),
            pl.BlockSpec((D, D), lambda i: (0, 0)),
            pl.BlockSpec((1, D), lambda i: (0, 0)),
            pl.BlockSpec((D, D), lambda i: (0, 0)),
            pl.BlockSpec((1, D), lambda i: (0, 0)),
            pl.BlockSpec((D, D), lambda i: (0, 0)),
            pl.BlockSpec((1, D), lambda i: (0, 0)),
        ],
        out_specs=[
            pl.BlockSpec((B, D), lambda i: (i, 0)),
            pl.BlockSpec((B, 2 * D), lambda i: (i, 0)),
        ],
        out_shape=[
            jax.ShapeDtypeStruct((NPAD, D), jnp.float32),
            jax.ShapeDtypeStruct((NPAD, 2 * D), jnp.float32),
        ],
    )(x, Wq, bq, Wk, bk, Wv, bv)


@functools.partial(
    pl.kernel,
    mesh=_mesh,
    out_type=[
        jax.ShapeDtypeStruct((EH, D), jnp.float32),
        jax.ShapeDtypeStruct((EH, 2 * D), jnp.float32),
    ],
    scratch_types=[
        pltpu.VMEM((2, G), jnp.int32),
        pltpu.VMEM((2, G), jnp.int32),
        pltpu.VMEM((2 * G, D), jnp.float32),
        pltpu.VMEM((2 * G, 2 * D), jnp.float32),
        pltpu.SemaphoreType.DMA,
        pltpu.SemaphoreType.DMA,
        pltpu.SemaphoreType.DMA,
        pltpu.SemaphoreType.DMA,
    ],
)
def _sc_gather(q_hbm, kv_hbm, src_hbm, dst_hbm, qg_hbm, kvg_hbm,
               sidx_v, didx_v, qrow_v, kvrow_v, semg0, semg1, semw0, semw1):
    wid = lax.axis_index("s") * 2 + lax.axis_index("c")
    base_w = wid * EPW
    semg = (semg0, semg1)
    semw = (semw0, semw1)

    def fetch(ci, slot):
        base = base_w + ci * G
        pltpu.sync_copy(src_hbm.at[pl.ds(base, G)], sidx_v.at[slot])
        pltpu.sync_copy(dst_hbm.at[pl.ds(base, G)], didx_v.at[slot])
        pltpu.async_copy(kv_hbm.at[sidx_v.at[slot]],
                         kvrow_v.at[pl.ds(slot * G, G)], semg[slot])
        pltpu.async_copy(q_hbm.at[didx_v.at[slot]],
                         qrow_v.at[pl.ds(slot * G, G)], semg[slot])

    def drain(ci, slot):
        base = base_w + ci * G
        pltpu.make_async_copy(kv_hbm.at[sidx_v.at[slot]],
                              kvrow_v.at[pl.ds(slot * G, G)],
                              semg[slot]).wait()
        pltpu.make_async_copy(q_hbm.at[didx_v.at[slot]],
                              qrow_v.at[pl.ds(slot * G, G)],
                              semg[slot]).wait()
        pltpu.async_copy(kvrow_v.at[pl.ds(slot * G, G)],
                         kvg_hbm.at[pl.ds(base, G)], semw[slot])
        pltpu.async_copy(qrow_v.at[pl.ds(slot * G, G)],
                         qg_hbm.at[pl.ds(base, G)], semw[slot])

    def wait_wb(ci, slot):
        base = base_w + ci * G
        pltpu.make_async_copy(kvrow_v.at[pl.ds(slot * G, G)],
                              kvg_hbm.at[pl.ds(base, G)], semw[slot]).wait()
        pltpu.make_async_copy(qrow_v.at[pl.ds(slot * G, G)],
                              qg_hbm.at[pl.ds(base, G)], semw[slot]).wait()

    fetch(0, 0)
    fetch(1, 1)

    @pl.loop(0, CHUNKS - 2, step=2)
    def _(ci):
        drain(ci, 0)
        wait_wb(ci, 0)
        fetch(ci + 2, 0)
        drain(ci + 1, 1)
        wait_wb(ci + 1, 1)
        fetch(ci + 3, 1)

    drain(CHUNKS - 2, 0)
    wait_wb(CHUNKS - 2, 0)
    drain(CHUNKS - 1, 1)
    wait_wb(CHUNKS - 1, 1)


def _edge_body(qg_ref, kvg_ref, ea_ref, we_ref, num_ref, ex_ref):
    q = qg_ref[...]
    k = kvg_ref[:, :D]
    v = kvg_ref[:, D:]
    e = jax.lax.dot_general(ea_ref[...], we_ref[...],
                            (((1,), (0,)), ((), ())), precision=_HI)
    k_j = k + e
    v_j = v + e
    p = q * k_j
    r = jax.lax.broadcasted_iota(jnp.int32, (D, D), 0) // C
    c2 = jax.lax.broadcasted_iota(jnp.int32, (D, D), 1) // C
    ss = (r == c2).astype(jnp.float32)
    alpha128 = jax.lax.dot_general(p, ss, (((1,), (0,)), ((), ())),
                                   precision=_HI)
    ex128 = jnp.exp(alpha128)
    num_ref[...] = ex128 * v_j
    r8 = jax.lax.broadcasted_iota(jnp.int32, (D, 16), 0) // C
    c8 = jax.lax.broadcasted_iota(jnp.int32, (D, 16), 1)
    s8 = ((r8 == c8) & (c8 < H)).astype(jnp.float32) * np.float32(1.0 / C)
    ex_ref[...] = jax.lax.dot_general(ex128, s8, (((1,), (0,)), ((), ())),
                                      precision=_HI)


def _edge_math(qg, kvg, ea, We):
    B = 2048
    return pl.pallas_call(
        _edge_body,
        grid=(qg.shape[0] // B,),
        in_specs=[
            pl.BlockSpec((B, D), lambda i: (i, 0)),
            pl.BlockSpec((B, 2 * D), lambda i: (i, 0)),
            pl.BlockSpec((B, ED), lambda i: (i, 0)),
            pl.BlockSpec((ED, D), lambda i: (0, 0)),
        ],
        out_specs=[
            pl.BlockSpec((B, D), lambda i: (i, 0)),
            pl.BlockSpec((B, 16), lambda i: (i, 0)),
        ],
        out_shape=[
            jax.ShapeDtypeStruct((qg.shape[0], D), jnp.float32),
            jax.ShapeDtypeStruct((qg.shape[0], 16), jnp.float32),
        ],
    )(qg, kvg, ea, We)


@functools.partial(
    pl.kernel,
    mesh=_mesh,
    out_type=[
        jax.ShapeDtypeStruct((2, ACCR, D), jnp.float32),
        jax.ShapeDtypeStruct((2, DROWS, D), jnp.float32),
    ],
    scratch_types=[
        pltpu.VMEM((2, GS), jnp.int32),
        pltpu.VMEM((2, GS), jnp.int32),
        pltpu.VMEM((64,), jnp.int32),
        pltpu.VMEM((2 * GS, D), jnp.float32),
        pltpu.VMEM((2 * GS, 16), jnp.float32),
        pltpu.VMEM((8, D), jnp.float32),
        pltpu.VMEM((DROWS, D), jnp.float32),
        pltpu.VMEM_SHARED((ACCR, D), jnp.float32),
        pltpu.VMEM_SHARED((DROWS, D), jnp.float32),
        pltpu.SemaphoreType.DMA,
        pltpu.SemaphoreType.DMA,
        pltpu.SemaphoreType.DMA,
        pltpu.SemaphoreType.DMA,
        pltpu.SemaphoreType.DMA,
    ],
    compiler_params=_cp,
)
def _sc_scatter(num_hbm, ex_hbm, dst_hbm, outn_hbm, outd_hbm, didx_v, lidx_v,
                midx_v, nrows_v, xrows_v, zbuf_v, den_v, accn_sh, accd_sh,
                semf0, semf1, semn0, semn1, semd):
    cid = lax.axis_index("c")
    sid = lax.axis_index("s")
    base_node = cid * NPC0
    limit = jnp.where(cid == 0, NPC0, ACCR)
    trash = jnp.where(cid == 0, NPC0 + 8, ACCR - 1)
    iota16 = lax.iota(jnp.int32, 16)

    @pl.loop(0, 8)
    def _(rr):
        for cc in range(0, D, 16):
            zbuf_v[rr, pl.ds(cc, 16)] = jnp.zeros((16,), jnp.float32)

    @pl.loop(0, DROWS)
    def _(rr):
        for cc in range(0, D, 16):
            den_v[rr, pl.ds(cc, 16)] = jnp.zeros((16,), jnp.float32)

    for j in range((NBLK + 15) // 16):
        blk = j * 16 + sid

        @pl.when(blk < NBLK)
        def _():
            pltpu.sync_copy(zbuf_v, accn_sh.at[pl.ds(blk * 8, 8)])
    for j in range((DROWS // 8 + 15) // 16):
        blk = j * 16 + sid

        @pl.when(blk < DROWS // 8)
        def _():
            pltpu.sync_copy(zbuf_v, accd_sh.at[pl.ds(blk * 8, 8)])
    plsc.subcore_barrier()

    semf = (semf0, semf1)
    semn = (semn0, semn1)

    def fetch(ci, slot):
        base = sid * EPS + ci * GS
        pltpu.async_copy(dst_hbm.at[pl.ds(base, GS)], didx_v.at[slot],
                         semf[slot])
        pltpu.async_copy(num_hbm.at[pl.ds(base, GS)],
                         nrows_v.at[pl.ds(slot * GS, GS)], semf[slot])
        pltpu.async_copy(ex_hbm.at[pl.ds(base, GS)],
                         xrows_v.at[pl.ds(slot * GS, GS)], semf[slot])

    def proc(ci, slot):
        base = sid * EPS + ci * GS
        pltpu.make_async_copy(dst_hbm.at[pl.ds(base, GS)], didx_v.at[slot],
                              semf[slot]).wait()
        pltpu.make_async_copy(num_hbm.at[pl.ds(base, GS)],
                              nrows_v.at[pl.ds(slot * GS, GS)],
                              semf[slot]).wait()
        pltpu.make_async_copy(ex_hbm.at[pl.ds(base, GS)],
                              xrows_v.at[pl.ds(slot * GS, GS)],
                              semf[slot]).wait()
        for i in range(GS // 16):
            d = didx_v[slot, pl.ds(i * 16, 16)]
            rel = d - base_node
            ok = (rel >= 0) & (rel < limit)
            lidx_v[slot, pl.ds(i * 16, 16)] = jnp.where(ok, rel, trash)
        a = pltpu.async_copy(nrows_v.at[pl.ds(slot * GS, GS)],
                             accn_sh.at[lidx_v.at[slot]], semn[slot],
                             add=True)

        @pl.loop(0, GS // 16)
        def _(i):
            lv = lidx_v[slot, pl.ds(i * 16, 16)]
            for j in range(16):
                f = lv[j] * 8
                fl = f + iota16
                row = lax.shift_right_logical(fl, 7)
                col = lax.bitwise_and(fl, 127)
                val = xrows_v[slot * GS + i * 16 + j, :]
                plsc.addupdate_scatter(den_v, [row, col], val)

        a.wait()

    fetch(0, 0)
    fetch(1, 1)

    @pl.loop(0, SCH - 2, step=2)
    def _(ci):
        proc(ci, 0)
        fetch(ci + 2, 0)
        proc(ci + 1, 1)
        fetch(ci + 3, 1)

    proc(SCH - 2, 0)
    proc(SCH - 1, 1)

    for c in range(DROWS // 64):
        for i in range(4):
            midx_v[pl.ds(i * 16, 16)] = iota16 + (c * 64 + i * 16)
        pltpu.async_copy(den_v.at[pl.ds(c * 64, 64)], accd_sh.at[midx_v],
                         semd, add=True).wait()

    plsc.subcore_barrier()
    for j in range((NBLK + 15) // 16):
        blk = j * 16 + sid

        @pl.when(blk < NBLK)
        def _():
            pltpu.sync_copy(accn_sh.at[pl.ds(blk * 8, 8)],
                            outn_hbm.at[cid].at[pl.ds(blk * 8, 8)])
    for j in range((DROWS // 8 + 15) // 16):
        blk = j * 16 + sid

        @pl.when(blk < DROWS // 8)
        def _():
            pltpu.sync_copy(accd_sh.at[pl.ds(blk * 8, 8)],
                            outd_hbm.at[cid].at[pl.ds(blk * 8, 8)])


def _tail_body(num0_ref, num1_ref, den80_ref, den81_ref, x_ref, wsk_ref,
               bsk_ref, g1_ref, b1n_ref, w1_ref, b1_ref, w2_ref, b2_ref,
               g2_ref, b2n_ref, o_ref):
    num = num0_ref[...] + num1_ref[...]
    den8 = den80_ref[...] + den81_ref[...]
    r8 = jax.lax.broadcasted_iota(jnp.int32, (8, D), 0)
    c8 = jax.lax.broadcasted_iota(jnp.int32, (8, D), 1) // C
    s8t = (r8 == c8).astype(jnp.float32)
    den128 = jax.lax.dot_general(den8, s8t, (((1,), (0,)), ((), ())),
                                 precision=_HI)
    agg = num / (den128 + np.float32(1e-16))
    x = x_ref[...]
    conv = agg + x @ wsk_ref[...] + bsk_ref[...]
    y = conv + x
    m = jnp.mean(y, axis=-1, keepdims=True)
    var = jnp.mean((y - m) ** 2, axis=-1, keepdims=True)
    h = (y - m) / jnp.sqrt(var + 1e-5) * g1_ref[...] + b1n_ref[...]
    z = h @ w1_ref[...] + b1_ref[...]
    gel = z * 0.5 * (1.0 + jax.lax.erf(z * np.float32(1.0 / np.sqrt(2.0))))
    f = gel @ w2_ref[...] + b2_ref[...]
    y2 = f + h
    m2 = jnp.mean(y2, axis=-1, keepdims=True)
    var2 = jnp.mean((y2 - m2) ** 2, axis=-1, keepdims=True)
    o_ref[...] = (y2 - m2) / jnp.sqrt(var2 + 1e-5) * g2_ref[...] + b2n_ref[...]


def _tail(num0, num1, den80, den81, x, Wskip, bskip, ln1_g, ln1_b, W1, b1,
          W2, b2, ln2_g, ln2_b):
    B = 1024
    return pl.pallas_call(
        _tail_body,
        grid=(NPAD // B,),
        in_specs=[
            pl.BlockSpec((B, D), lambda i: (i, 0)),
            pl.BlockSpec((B, D), lambda i: (i, 0)),
            pl.BlockSpec((B, 8), lambda i: (i, 0)),
            pl.BlockSpec((B, 8), lambda i: (i, 0)),
            pl.BlockSpec((B, D), lambda i: (i, 0)),
            pl.BlockSpec((D, D), lambda i: (0, 0)),
            pl.BlockSpec((1, D), lambda i: (0, 0)),
            pl.BlockSpec((1, D), lambda i: (0, 0)),
            pl.BlockSpec((1, D), lambda i: (0, 0)),
            pl.BlockSpec((D, 4 * D), lambda i: (0, 0)),
            pl.BlockSpec((1, 4 * D), lambda i: (0, 0)),
            pl.BlockSpec((4 * D, D), lambda i: (0, 0)),
            pl.BlockSpec((1, D), lambda i: (0, 0)),
            pl.BlockSpec((1, D), lambda i: (0, 0)),
            pl.BlockSpec((1, D), lambda i: (0, 0)),
        ],
        out_specs=pl.BlockSpec((B, D), lambda i: (i, 0)),
        out_shape=jax.ShapeDtypeStruct((NPAD, D), jnp.float32),
    )(num0, num1, den80, den81, x, Wskip, bskip, ln1_g, ln1_b,
      W1, b1, W2, b2, ln2_g, ln2_b)


def kernel(x, edge_index, edge_attr, Wq, bq, Wk, bk, Wv, bv, We, Wskip, bskip,
           ln1_g, ln1_b, ln2_g, ln2_b, W1, b1, W2, b2):
    xp = jnp.pad(x, ((0, NPAD - N), (0, 0)))
    q, kv = _qkv(xp, Wq, bq.reshape(1, D), Wk, bk.reshape(1, D),
                 Wv, bv.reshape(1, D))

    src = edge_index[0]
    dst = edge_index[1]
    srcp = jnp.concatenate([src, jnp.zeros((EPAD - E,), src.dtype)]).astype(jnp.int32)
    dstp = jnp.concatenate([dst, jnp.full((EPAD - E,), N, dst.dtype)]).astype(jnp.int32)

    eap = jnp.pad(edge_attr, ((0, EPAD - E), (0, 0)))

    nums = []
    den8s = []
    for h in range(2):
        sl = slice(h * EH, (h + 1) * EH)
        qg, kvg = _sc_gather(q, kv, srcp[sl], dstp[sl])
        nume, exe = _edge_math(qg, kvg, eap[sl], We)
        pn, pd = _sc_scatter(nume, exe, dstp[sl])
        nums.append(
            jnp.pad(jnp.concatenate([pn[0, :NPC0], pn[1]], axis=0),
                    ((0, NPAD - NPC0 - ACCR), (0, 0))))
        pd8 = pd.reshape(2, DROWS * D // 8, 8)
        den8s.append(
            jnp.pad(jnp.concatenate([pd8[0, :NPC0], pd8[1, :ACCR]], axis=0),
                    ((0, NPAD - NPC0 - ACCR), (0, 0))))

    out = _tail(nums[0], nums[1], den8s[0], den8s[1], xp, Wskip,
                bskip.reshape(1, D),
                ln1_g.reshape(1, D), ln1_b.reshape(1, D),
                W1, b1.reshape(1, 4 * D), W2, b2.reshape(1, D),
                ln2_g.reshape(1, D), ln2_b.reshape(1, D))
    return out[:N]
